# Initial kernel scaffold; baseline (speedup 1.0000x reference)
import jax
import jax.numpy as jnp
from jax.experimental import pallas as pl


def kernel(v, pos, edges, offsets_real, W, b, bn1_g, bn1_b, bn2_g, bn2_b):
    raise NotImplementedError("write your pallas kernel here")


# speedup vs baseline: 1.0884x; 1.0884x over previous
import functools

import jax
import jax.numpy as jnp
from jax import lax
from jax.experimental import pallas as pl
from jax.experimental.pallas import tpu as pltpu
from jax.experimental.pallas import tpu_sc as plsc

N = 10000
E = 320000
H = 64
L = 3
NBR = 41
NBRP = 48
NW = 32
EW = E // NW
C = 80
NCHUNK = EW // C
NP = 10240
RPT = NP // 16
H2 = H // 2
EC = E // 16
NCHUNK2 = EC // C
INV_E = 1.0 / E
EPS = 1e-5

_MESH = dict(core_axis_name="c", subcore_axis_name="s")


@functools.partial(
    pl.kernel,
    out_type=jax.ShapeDtypeStruct((E,), jnp.float32),
    mesh=plsc.VectorSubcoreMesh(**_MESH),
    scratch_types=[
        pltpu.VMEM((C,), jnp.int32),
        pltpu.VMEM((C,), jnp.int32),
        pltpu.VMEM((C,), jnp.float32),
        pltpu.VMEM((C,), jnp.float32),
        pltpu.VMEM((C,), jnp.float32),
        pltpu.VMEM((C,), jnp.float32),
        pltpu.VMEM((C,), jnp.float32),
        pltpu.VMEM((C,), jnp.float32),
        pltpu.VMEM((C,), jnp.float32),
        pltpu.VMEM((C,), jnp.float32),
        pltpu.VMEM((C,), jnp.float32),
        pltpu.VMEM((C,), jnp.float32),
        pltpu.SemaphoreType.DMA,
    ],
)
def _dist2(px_hbm, py_hbm, pz_hbm, dst_hbm, src_hbm, ox_hbm, oy_hbm, oz_hbm,
           out_hbm,
           idxd, idxs, gdx, gdy, gdz, gsx, gsy, gsz, oxv, oyv, ozv, d2v, sem):
    wid = lax.axis_index("s") * 2 + lax.axis_index("c")
    wbase = wid * EW

    def chunk_body(ci, _):
        base = pl.multiple_of(wbase + ci * C, 8)
        pltpu.sync_copy(dst_hbm.at[pl.ds(base, C)], idxd)
        pltpu.sync_copy(src_hbm.at[pl.ds(base, C)], idxs)
        cps = [
            pltpu.async_copy(px_hbm.at[idxd], gdx, sem),
            pltpu.async_copy(py_hbm.at[idxd], gdy, sem),
            pltpu.async_copy(pz_hbm.at[idxd], gdz, sem),
            pltpu.async_copy(px_hbm.at[idxs], gsx, sem),
            pltpu.async_copy(py_hbm.at[idxs], gsy, sem),
            pltpu.async_copy(pz_hbm.at[idxs], gsz, sem),
        ]
        pltpu.sync_copy(ox_hbm.at[pl.ds(base, C)], oxv)
        pltpu.sync_copy(oy_hbm.at[pl.ds(base, C)], oyv)
        pltpu.sync_copy(oz_hbm.at[pl.ds(base, C)], ozv)
        for cp in cps:
            cp.wait()

        def body(g, _):
            s = pl.ds(g * 16, 16)
            dx = gdx[s] + oxv[s] - gsx[s]
            dy = gdy[s] + oyv[s] - gsy[s]
            dz = gdz[s] + ozv[s] - gsz[s]
            d2v[s] = dx * dx + dy * dy + dz * dz
            return 0

        lax.fori_loop(0, C // 16, body, 0)
        pltpu.sync_copy(d2v, out_hbm.at[pl.ds(base, C)])
        return 0

    lax.fori_loop(0, NCHUNK, chunk_body, 0)


def _nf_body(d2_ref, wn_ref, bias_ref, nf0_ref, nf1_ref, nf2_ref):
    d = jnp.sqrt(d2_ref[...])
    cent = lax.broadcasted_iota(jnp.int32, (1, NBRP), 1).astype(jnp.float32) * 0.2
    g = jnp.exp(-((d - cent) ** 2) * 25.0)
    o = jnp.dot(g, wn_ref[...], preferred_element_type=jnp.float32)
    o = o + bias_ref[...]
    nf0_ref[...] = o[:, 0:128]
    nf1_ref[...] = o[:, 128:256]
    nf2_ref[...] = o[:, 256:384]


_EB = 512


def _nf(d2c, wn, bias):
    return pl.pallas_call(
        _nf_body,
        grid=(E // _EB,),
        in_specs=[
            pl.BlockSpec((_EB, 1), lambda i: (i, 0)),
            pl.BlockSpec((NBRP, 3 * 128), lambda i: (0, 0)),
            pl.BlockSpec((1, 3 * 128), lambda i: (0, 0)),
        ],
        out_specs=[
            pl.BlockSpec((_EB, 128), lambda i: (i, 0)),
            pl.BlockSpec((_EB, 128), lambda i: (i, 0)),
            pl.BlockSpec((_EB, 128), lambda i: (i, 0)),
        ],
        out_shape=[jax.ShapeDtypeStruct((E, 128), jnp.float32)] * 3,
    )(d2c, wn, bias)


def _aproj_body(atom_ref, wd_ref, ws_ref, ad_ref, as_ref):
    x = atom_ref[...]
    ad_ref[...] = jnp.dot(x, wd_ref[...], preferred_element_type=jnp.float32)
    as_ref[...] = jnp.dot(x, ws_ref[...], preferred_element_type=jnp.float32)


def _aproj(atom, wd, ws):
    nb = 2000
    return pl.pallas_call(
        _aproj_body,
        grid=(N // nb,),
        in_specs=[
            pl.BlockSpec((nb, H), lambda i: (i, 0)),
            pl.BlockSpec((H, 128), lambda i: (0, 0)),
            pl.BlockSpec((H, 128), lambda i: (0, 0)),
        ],
        out_specs=[
            pl.BlockSpec((nb, 128), lambda i: (i, 0)),
            pl.BlockSpec((nb, 128), lambda i: (i, 0)),
        ],
        out_shape=[jax.ShapeDtypeStruct((N, 128), jnp.float32)] * 2,
    )(atom, wd, ws)


@functools.partial(
    pl.kernel,
    out_type=[
        jax.ShapeDtypeStruct((E, 128), jnp.float32),
        jax.ShapeDtypeStruct((NW, 2, 8, 16), jnp.float32),
    ],
    mesh=plsc.VectorSubcoreMesh(**_MESH),
    scratch_types=[
        pltpu.VMEM((C,), jnp.int32),
        pltpu.VMEM((C,), jnp.int32),
        pltpu.VMEM((C, 128), jnp.float32),
        pltpu.VMEM((C, 128), jnp.float32),
        pltpu.VMEM((C, 128), jnp.float32),
        pltpu.VMEM((C, 128), jnp.float32),
        pltpu.VMEM((8, 16), jnp.float32),
        pltpu.VMEM((8, 16), jnp.float32),
        pltpu.SemaphoreType.DMA,
        pltpu.SemaphoreType.DMA,
    ],
)
def _pass1(dst_hbm, src_hbm, ad_hbm, as_hbm, nf_hbm, tot_hbm, st_hbm,
           idxd, idxs, gd, gs, nfv, totv, sacc, qacc, semd, sems):
    wid = lax.axis_index("s") * 2 + lax.axis_index("c")
    z = jnp.zeros((16,), jnp.float32)
    for k in range(8):
        sacc[k] = z
        qacc[k] = z
    wbase = wid * EW

    def chunk_body(ci, _):
        base = pl.multiple_of(wbase + ci * C, 8)
        pltpu.sync_copy(dst_hbm.at[pl.ds(base, C)], idxd)
        pltpu.sync_copy(src_hbm.at[pl.ds(base, C)], idxs)
        cpd = pltpu.async_copy(ad_hbm.at[idxd], gd, semd)
        cps = pltpu.async_copy(as_hbm.at[idxs], gs, sems)
        pltpu.sync_copy(nf_hbm.at[pl.ds(base, C)], nfv)
        cpd.wait()
        cps.wait()

        def edge_body(j, _):
            for k in range(8):
                s = pl.ds(k * 16, 16)
                t = gd[j, s] + gs[j, s] + nfv[j, s]
                totv[j, s] = t
                plsc.addupdate(sacc.at[k], t)
                plsc.addupdate(qacc.at[k], t * t)
            return 0

        lax.fori_loop(0, C, edge_body, 0)
        pltpu.sync_copy(totv, tot_hbm.at[pl.ds(base, C)])
        return 0

    lax.fori_loop(0, NCHUNK, chunk_body, 0)
    pltpu.sync_copy(sacc, st_hbm.at[wid, 0])
    pltpu.sync_copy(qacc, st_hbm.at[wid, 1])


def _pass2a_body(tot_ref, sums_ref, sqs_ref, g_ref, bb_ref, msg_ref):
    s = jnp.sum(sums_ref[...], axis=0, keepdims=True)
    q = jnp.sum(sqs_ref[...], axis=0, keepdims=True)
    m = s * INV_E
    var = q * INV_E - m * m
    al = g_ref[...] * lax.rsqrt(var + EPS)
    be = bb_ref[...] - m * al
    y = tot_ref[...] * al + be
    f = y[:, :H]
    c = y[:, H:]
    sig = 1.0 / (1.0 + jnp.exp(-f))
    sp = jnp.maximum(c, 0.0) + jnp.log(1.0 + jnp.exp(-jnp.abs(c)))
    msg = sig * sp
    msg_ref[0] = msg[:, :H2]
    msg_ref[1] = msg[:, H2:]


def _pass2a(tot, sums, sqs, g, bb):
    return pl.pallas_call(
        _pass2a_body,
        grid=(E // _EB,),
        in_specs=[
            pl.BlockSpec((_EB, 128), lambda i: (i, 0)),
            pl.BlockSpec((NW, 128), lambda i: (0, 0)),
            pl.BlockSpec((NW, 128), lambda i: (0, 0)),
            pl.BlockSpec((1, 128), lambda i: (0, 0)),
            pl.BlockSpec((1, 128), lambda i: (0, 0)),
        ],
        out_specs=pl.BlockSpec((2, _EB, H2), lambda i: (0, i, 0)),
        out_shape=jax.ShapeDtypeStruct((2, E, H2), jnp.float32),
    )(tot, sums, sqs, g, bb)


@functools.partial(
    pl.kernel,
    out_type=jax.ShapeDtypeStruct((2, NP, H2), jnp.float32),
    mesh=plsc.VectorSubcoreMesh(**_MESH),
    scratch_types=[
        pltpu.VMEM((C,), jnp.int32),
        pltpu.VMEM((C, H2), jnp.float32),
        pltpu.VMEM((C, 128), jnp.float32),
        pltpu.VMEM_SHARED((NP, H2), jnp.float32),
    ],
)
def _scatter(dst_hbm, msg_hbm, znp_hbm, out_hbm, idxv, msgv, mwide, aggr_sh):
    cid = lax.axis_index("c")
    sid = lax.axis_index("s")

    @pl.when(sid == 0)
    def _():
        pltpu.sync_copy(znp_hbm.at[cid], aggr_sh)

    plsc.subcore_barrier()
    wbase = sid * EC

    def chunk_body(ci, _):
        base = pl.multiple_of(wbase + ci * C, 8)
        pltpu.sync_copy(dst_hbm.at[pl.ds(base, C)], idxv)
        pltpu.sync_copy(msg_hbm.at[cid, pl.ds(base, C)], aggr_sh.at[idxv],
                        add=True)
        return 0

    lax.fori_loop(0, NCHUNK2, chunk_body, 0)
    plsc.subcore_barrier()
    pltpu.sync_copy(aggr_sh.at[pl.ds(sid * RPT, RPT)],
                    out_hbm.at[cid, pl.ds(sid * RPT, RPT)])


def _final_body(ap_ref, atom_ref, g2_ref, b2_ref, out_ref):
    a = jnp.concatenate([ap_ref[0, :N, :], ap_ref[1, :N, :]], axis=1)
    m = jnp.mean(a, axis=0, keepdims=True)
    var = jnp.mean((a - m) ** 2, axis=0, keepdims=True)
    ag = (a - m) * lax.rsqrt(var + EPS) * g2_ref[...] + b2_ref[...]
    x = atom_ref[...] + ag
    out_ref[...] = jnp.maximum(x, 0.0) + jnp.log(1.0 + jnp.exp(-jnp.abs(x)))


def _final(ap, atom, g2, b2):
    return pl.pallas_call(
        _final_body,
        out_shape=jax.ShapeDtypeStruct((N, H), jnp.float32),
    )(ap, atom, g2, b2)


def kernel(v, pos, edges, offsets_real, W, b, bn1_g, bn1_b, bn2_g, bn2_b):
    src = edges[0].astype(jnp.int32)
    dst = edges[1].astype(jnp.int32)
    offt = offsets_real.T

    post = pos.T
    d2 = _dist2(post[0], post[1], post[2], dst, src, offt[0], offt[1], offt[2])
    d2c = d2.reshape(E, 1)

    wn = jnp.concatenate([W[l, 2 * H:] for l in range(L)], axis=1)
    wn = jnp.pad(wn, ((0, NBRP - NBR), (0, 0)))
    bias_all = jnp.concatenate([b[l] for l in range(L)])[None, :]
    nfs = _nf(d2c, wn, bias_all)

    atom = v
    for l in range(L):
        ad, as_ = _aproj(atom, W[l, :H], W[l, H:2 * H])
        tot, st = _pass1(dst, src, ad, as_, nfs[l])
        str_ = st.reshape(NW, 2, 128)
        sums = str_[:, 0]
        sqs = str_[:, 1]
        msg = _pass2a(tot, sums, sqs,
                      bn1_g[l][None, :], bn1_b[l][None, :])
        m2 = jnp.concatenate([msg[0], msg[1]], axis=1)
        aggr = jax.ops.segment_sum(m2, dst, num_segments=N)
        ap = jnp.zeros((2, NP, H2), jnp.float32)
        ap = ap.at[0, :N].set(aggr[:, :H2]).at[1, :N].set(aggr[:, H2:])
        atom = _final(ap, atom, bn2_g[l][None, :], bn2_b[l][None, :])
    return atom


# speedup vs baseline: 1.1689x; 1.0740x over previous
import functools

import jax
import jax.numpy as jnp
from jax import lax
from jax.experimental import pallas as pl
from jax.experimental.pallas import tpu as pltpu
from jax.experimental.pallas import tpu_sc as plsc

N = 10000
E = 320000
H = 64
L = 3
NBR = 41
NBRP = 48
NW = 32
EW = E // NW
C = 80
NCHUNK = EW // C
NP = 10240
RPT = NP // 16
H2 = H // 2
EC = E // 16
NCHUNK2 = EC // C
INV_E = 1.0 / E
EPS = 1e-5

_MESH = dict(core_axis_name="c", subcore_axis_name="s")


@functools.partial(
    pl.kernel,
    out_type=jax.ShapeDtypeStruct((E,), jnp.float32),
    mesh=plsc.VectorSubcoreMesh(**_MESH),
    scratch_types=[
        pltpu.VMEM((C,), jnp.int32),
        pltpu.VMEM((C,), jnp.int32),
        pltpu.VMEM((C,), jnp.float32),
        pltpu.VMEM((C,), jnp.float32),
        pltpu.VMEM((C,), jnp.float32),
        pltpu.VMEM((C,), jnp.float32),
        pltpu.VMEM((C,), jnp.float32),
        pltpu.VMEM((C,), jnp.float32),
        pltpu.VMEM((C,), jnp.float32),
        pltpu.VMEM((C,), jnp.float32),
        pltpu.VMEM((C,), jnp.float32),
        pltpu.VMEM((C,), jnp.float32),
        pltpu.SemaphoreType.DMA,
    ],
)
def _dist2(px_hbm, py_hbm, pz_hbm, dst_hbm, src_hbm, ox_hbm, oy_hbm, oz_hbm,
           out_hbm,
           idxd, idxs, gdx, gdy, gdz, gsx, gsy, gsz, oxv, oyv, ozv, d2v, sem):
    wid = lax.axis_index("s") * 2 + lax.axis_index("c")
    wbase = wid * EW

    def chunk_body(ci, _):
        base = pl.multiple_of(wbase + ci * C, 8)
        pltpu.sync_copy(dst_hbm.at[pl.ds(base, C)], idxd)
        pltpu.sync_copy(src_hbm.at[pl.ds(base, C)], idxs)
        cps = [
            pltpu.async_copy(px_hbm.at[idxd], gdx, sem),
            pltpu.async_copy(py_hbm.at[idxd], gdy, sem),
            pltpu.async_copy(pz_hbm.at[idxd], gdz, sem),
            pltpu.async_copy(px_hbm.at[idxs], gsx, sem),
            pltpu.async_copy(py_hbm.at[idxs], gsy, sem),
            pltpu.async_copy(pz_hbm.at[idxs], gsz, sem),
        ]
        pltpu.sync_copy(ox_hbm.at[pl.ds(base, C)], oxv)
        pltpu.sync_copy(oy_hbm.at[pl.ds(base, C)], oyv)
        pltpu.sync_copy(oz_hbm.at[pl.ds(base, C)], ozv)
        for cp in cps:
            cp.wait()

        def body(g, _):
            s = pl.ds(g * 16, 16)
            dx = gdx[s] + oxv[s] - gsx[s]
            dy = gdy[s] + oyv[s] - gsy[s]
            dz = gdz[s] + ozv[s] - gsz[s]
            d2v[s] = dx * dx + dy * dy + dz * dz
            return 0

        lax.fori_loop(0, C // 16, body, 0)
        pltpu.sync_copy(d2v, out_hbm.at[pl.ds(base, C)])
        return 0

    lax.fori_loop(0, NCHUNK, chunk_body, 0)


def _nf_body(d2_ref, wn_ref, bias_ref, nf0_ref, nf1_ref, nf2_ref):
    d = jnp.sqrt(d2_ref[...])
    cent = lax.broadcasted_iota(jnp.int32, (1, NBRP), 1).astype(jnp.float32) * 0.2
    g = jnp.exp(-((d - cent) ** 2) * 25.0)
    o = jnp.dot(g, wn_ref[...], preferred_element_type=jnp.float32)
    o = o + bias_ref[...]
    nf0_ref[...] = o[:, 0:128]
    nf1_ref[...] = o[:, 128:256]
    nf2_ref[...] = o[:, 256:384]


_EB = 512


def _nf(d2c, wn, bias):
    return pl.pallas_call(
        _nf_body,
        grid=(E // _EB,),
        in_specs=[
            pl.BlockSpec((_EB, 1), lambda i: (i, 0)),
            pl.BlockSpec((NBRP, 3 * 128), lambda i: (0, 0)),
            pl.BlockSpec((1, 3 * 128), lambda i: (0, 0)),
        ],
        out_specs=[
            pl.BlockSpec((_EB, 128), lambda i: (i, 0)),
            pl.BlockSpec((_EB, 128), lambda i: (i, 0)),
            pl.BlockSpec((_EB, 128), lambda i: (i, 0)),
        ],
        out_shape=[jax.ShapeDtypeStruct((E, 128), jnp.float32)] * 3,
    )(d2c, wn, bias)


def _aproj_body(atom_ref, wd_ref, ws_ref, ad_ref, as_ref):
    x = atom_ref[...]
    ad_ref[...] = jnp.dot(x, wd_ref[...], preferred_element_type=jnp.float32)
    as_ref[...] = jnp.dot(x, ws_ref[...], preferred_element_type=jnp.float32)


def _aproj(atom, wd, ws):
    nb = 2000
    return pl.pallas_call(
        _aproj_body,
        grid=(N // nb,),
        in_specs=[
            pl.BlockSpec((nb, H), lambda i: (i, 0)),
            pl.BlockSpec((H, 128), lambda i: (0, 0)),
            pl.BlockSpec((H, 128), lambda i: (0, 0)),
        ],
        out_specs=[
            pl.BlockSpec((nb, 128), lambda i: (i, 0)),
            pl.BlockSpec((nb, 128), lambda i: (i, 0)),
        ],
        out_shape=[jax.ShapeDtypeStruct((N, 128), jnp.float32)] * 2,
    )(atom, wd, ws)


@functools.partial(
    pl.kernel,
    out_type=[
        jax.ShapeDtypeStruct((E, 128), jnp.float32),
        jax.ShapeDtypeStruct((NW, 2, 8, 16), jnp.float32),
    ],
    mesh=plsc.VectorSubcoreMesh(**_MESH),
    scratch_types=[
        pltpu.VMEM((2, C), jnp.int32),
        pltpu.VMEM((2, C), jnp.int32),
        pltpu.VMEM((2, C, 128), jnp.float32),
        pltpu.VMEM((2, C, 128), jnp.float32),
        pltpu.VMEM((2, C, 128), jnp.float32),
        pltpu.VMEM((2, C, 128), jnp.float32),
        pltpu.VMEM((8, 16), jnp.float32),
        pltpu.VMEM((8, 16), jnp.float32),
        pltpu.SemaphoreType.DMA,
        pltpu.SemaphoreType.DMA,
        pltpu.SemaphoreType.DMA,
        pltpu.SemaphoreType.DMA,
        pltpu.SemaphoreType.DMA,
        pltpu.SemaphoreType.DMA,
    ],
)
def _pass1(dst_hbm, src_hbm, ad_hbm, as_hbm, nf_hbm, tot_hbm, st_hbm,
           idxd, idxs, gd, gs, nfv, totv, sacc, qacc,
           si0, si1, sg0, sg1, st0, st1):
    wid = lax.axis_index("s") * 2 + lax.axis_index("c")
    si = [si0, si1]
    sg = [sg0, sg1]
    st = [st0, st1]
    z = jnp.zeros((16,), jnp.float32)
    for k in range(8):
        sacc[k] = z
        qacc[k] = z
    wbase = wid * EW

    def bofs(c):
        return pl.multiple_of(wbase + c * C, 8)

    def prefetch(c, s):
        pltpu.async_copy(dst_hbm.at[pl.ds(bofs(c), C)], idxd.at[s], si[s])
        pltpu.async_copy(src_hbm.at[pl.ds(bofs(c), C)], idxs.at[s], si[s])

    def launch(c, s):
        pltpu.make_async_copy(dst_hbm.at[pl.ds(0, C)], idxd.at[s], si[s]).wait()
        pltpu.make_async_copy(src_hbm.at[pl.ds(0, C)], idxs.at[s], si[s]).wait()
        pltpu.async_copy(ad_hbm.at[idxd.at[s]], gd.at[s], sg[s])
        pltpu.async_copy(as_hbm.at[idxs.at[s]], gs.at[s], sg[s])
        pltpu.async_copy(nf_hbm.at[pl.ds(bofs(c), C)], nfv.at[s], sg[s])

    def consume_wait(s):
        pltpu.make_async_copy(nf_hbm.at[pl.ds(0, C)], gd.at[s], sg[s]).wait()
        pltpu.make_async_copy(nf_hbm.at[pl.ds(0, C)], gs.at[s], sg[s]).wait()
        pltpu.make_async_copy(nf_hbm.at[pl.ds(0, C)], nfv.at[s], sg[s]).wait()

    def compute_store(c, s, first):
        @pl.when(jnp.logical_not(first))
        def _():
            pltpu.make_async_copy(totv.at[s], tot_hbm.at[pl.ds(0, C)],
                                  st[s]).wait()

        def edge_body(j, _):
            for k in range(8):
                sl = pl.ds(k * 16, 16)
                t = gd[s, j, sl] + gs[s, j, sl] + nfv[s, j, sl]
                totv[s, j, sl] = t
                plsc.addupdate(sacc.at[k], t)
                plsc.addupdate(qacc.at[k], t * t)
            return 0

        lax.fori_loop(0, C, edge_body, 0)
        pltpu.async_copy(totv.at[s], tot_hbm.at[pl.ds(bofs(c), C)], st[s])

    prefetch(0, 0)
    launch(0, 0)
    prefetch(1, 1)

    def body(i, _):
        c0 = i * 2
        c1 = c0 + 1
        launch(c1, 1)
        consume_wait(0)
        prefetch(c0 + 2, 0)
        compute_store(c0, 0, i == 0)
        consume_wait(1)

        @pl.when(i < (NCHUNK - 3) // 2)
        def _():
            prefetch(c1 + 2, 1)

        compute_store(c1, 1, i == 0)
        launch(c0 + 2, 0)
        return 0

    lax.fori_loop(0, (NCHUNK - 1) // 2, body, 0)
    consume_wait(0)
    compute_store(NCHUNK - 1, 0, False)
    pltpu.make_async_copy(totv.at[0], tot_hbm.at[pl.ds(0, C)], st[0]).wait()
    pltpu.make_async_copy(totv.at[1], tot_hbm.at[pl.ds(0, C)], st[1]).wait()
    pltpu.sync_copy(sacc, st_hbm.at[wid, 0])
    pltpu.sync_copy(qacc, st_hbm.at[wid, 1])


def _pass2a_body(tot_ref, sums_ref, sqs_ref, g_ref, bb_ref, msg_ref):
    s = jnp.sum(sums_ref[...], axis=0, keepdims=True)
    q = jnp.sum(sqs_ref[...], axis=0, keepdims=True)
    m = s * INV_E
    var = q * INV_E - m * m
    al = g_ref[...] * lax.rsqrt(var + EPS)
    be = bb_ref[...] - m * al
    y = tot_ref[...] * al + be
    f = y[:, :H]
    c = y[:, H:]
    sig = 1.0 / (1.0 + jnp.exp(-f))
    sp = jnp.maximum(c, 0.0) + jnp.log(1.0 + jnp.exp(-jnp.abs(c)))
    msg = sig * sp
    msg_ref[0] = msg[:, :H2]
    msg_ref[1] = msg[:, H2:]


def _pass2a(tot, sums, sqs, g, bb):
    return pl.pallas_call(
        _pass2a_body,
        grid=(E // _EB,),
        in_specs=[
            pl.BlockSpec((_EB, 128), lambda i: (i, 0)),
            pl.BlockSpec((NW, 128), lambda i: (0, 0)),
            pl.BlockSpec((NW, 128), lambda i: (0, 0)),
            pl.BlockSpec((1, 128), lambda i: (0, 0)),
            pl.BlockSpec((1, 128), lambda i: (0, 0)),
        ],
        out_specs=pl.BlockSpec((2, _EB, H2), lambda i: (0, i, 0)),
        out_shape=jax.ShapeDtypeStruct((2, E, H2), jnp.float32),
    )(tot, sums, sqs, g, bb)


@functools.partial(
    pl.kernel,
    out_type=jax.ShapeDtypeStruct((2, NP, H2), jnp.float32),
    mesh=plsc.VectorSubcoreMesh(**_MESH),
    scratch_types=[
        pltpu.VMEM((C,), jnp.int32),
        pltpu.VMEM((C, H2), jnp.float32),
        pltpu.VMEM((C, 128), jnp.float32),
        pltpu.VMEM_SHARED((NP, H2), jnp.float32),
    ],
)
def _scatter(dst_hbm, msg_hbm, znp_hbm, out_hbm, idxv, msgv, mwide, aggr_sh):
    cid = lax.axis_index("c")
    sid = lax.axis_index("s")

    @pl.when(sid == 0)
    def _():
        pltpu.sync_copy(znp_hbm.at[cid], aggr_sh)

    plsc.subcore_barrier()
    wbase = sid * EC

    def chunk_body(ci, _):
        base = pl.multiple_of(wbase + ci * C, 8)
        pltpu.sync_copy(dst_hbm.at[pl.ds(base, C)], idxv)
        pltpu.sync_copy(msg_hbm.at[cid, pl.ds(base, C)], aggr_sh.at[idxv],
                        add=True)
        return 0

    lax.fori_loop(0, NCHUNK2, chunk_body, 0)
    plsc.subcore_barrier()
    pltpu.sync_copy(aggr_sh.at[pl.ds(sid * RPT, RPT)],
                    out_hbm.at[cid, pl.ds(sid * RPT, RPT)])


def _final_body(ap_ref, atom_ref, g2_ref, b2_ref, out_ref):
    a = jnp.concatenate([ap_ref[0, :N, :], ap_ref[1, :N, :]], axis=1)
    m = jnp.mean(a, axis=0, keepdims=True)
    var = jnp.mean((a - m) ** 2, axis=0, keepdims=True)
    ag = (a - m) * lax.rsqrt(var + EPS) * g2_ref[...] + b2_ref[...]
    x = atom_ref[...] + ag
    out_ref[...] = jnp.maximum(x, 0.0) + jnp.log(1.0 + jnp.exp(-jnp.abs(x)))


def _final(ap, atom, g2, b2):
    return pl.pallas_call(
        _final_body,
        out_shape=jax.ShapeDtypeStruct((N, H), jnp.float32),
    )(ap, atom, g2, b2)


def kernel(v, pos, edges, offsets_real, W, b, bn1_g, bn1_b, bn2_g, bn2_b):
    src = edges[0].astype(jnp.int32)
    dst = edges[1].astype(jnp.int32)
    offt = offsets_real.T

    post = pos.T
    d2 = _dist2(post[0], post[1], post[2], dst, src, offt[0], offt[1], offt[2])
    d2c = d2.reshape(E, 1)

    wn = jnp.concatenate([W[l, 2 * H:] for l in range(L)], axis=1)
    wn = jnp.pad(wn, ((0, NBRP - NBR), (0, 0)))
    bias_all = jnp.concatenate([b[l] for l in range(L)])[None, :]
    nfs = _nf(d2c, wn, bias_all)

    atom = v
    for l in range(L):
        ad, as_ = _aproj(atom, W[l, :H], W[l, H:2 * H])
        tot, st = _pass1(dst, src, ad, as_, nfs[l])
        str_ = st.reshape(NW, 2, 128)
        sums = str_[:, 0]
        sqs = str_[:, 1]
        msg = _pass2a(tot, sums, sqs,
                      bn1_g[l][None, :], bn1_b[l][None, :])
        m2 = jnp.concatenate([msg[0], msg[1]], axis=1)
        aggr = jax.ops.segment_sum(m2, dst, num_segments=N)
        ap = jnp.zeros((2, NP, H2), jnp.float32)
        ap = ap.at[0, :N].set(aggr[:, :H2]).at[1, :N].set(aggr[:, H2:])
        atom = _final(ap, atom, bn2_g[l][None, :], bn2_b[l][None, :])
    return atom


# speedup vs baseline: 1.1936x; 1.0211x over previous
import functools

import jax
import jax.numpy as jnp
from jax import lax
from jax.experimental import pallas as pl
from jax.experimental.pallas import tpu as pltpu
from jax.experimental.pallas import tpu_sc as plsc

N = 10000
E = 320000
H = 64
L = 3
NBR = 41
NBRP = 48
NW = 32
EW = E // NW
C = 80
SD = 2000
NCHUNK = EW // C
NP = 10240
RPT = NP // 16
H2 = H // 2
EC = E // 16
NCHUNK2 = EC // C
INV_E = 1.0 / E
EPS = 1e-5

_MESH = dict(core_axis_name="c", subcore_axis_name="s")


@functools.partial(
    pl.kernel,
    out_type=jax.ShapeDtypeStruct((E,), jnp.float32),
    mesh=plsc.VectorSubcoreMesh(**_MESH),
    scratch_types=[
        pltpu.VMEM((SD,), jnp.int32),
        pltpu.VMEM((SD,), jnp.int32),
        pltpu.VMEM((SD,), jnp.float32),
        pltpu.VMEM((SD,), jnp.float32),
        pltpu.VMEM((SD,), jnp.float32),
        pltpu.VMEM((SD,), jnp.float32),
        pltpu.VMEM((SD,), jnp.float32),
        pltpu.VMEM((SD,), jnp.float32),
        pltpu.VMEM((SD,), jnp.float32),
        pltpu.VMEM((SD,), jnp.float32),
        pltpu.VMEM((SD,), jnp.float32),
        pltpu.VMEM((SD,), jnp.float32),
        pltpu.SemaphoreType.DMA,
    ],
)
def _dist2(px_hbm, py_hbm, pz_hbm, dst_hbm, src_hbm, ox_hbm, oy_hbm, oz_hbm,
           out_hbm,
           idxd, idxs, gdx, gdy, gdz, gsx, gsy, gsz, oxv, oyv, ozv, d2v, sem):
    wid = lax.axis_index("s") * 2 + lax.axis_index("c")
    wbase = wid * EW

    def chunk_body(ci, _):
        base = pl.multiple_of(wbase + ci * SD, 8)
        pltpu.sync_copy(dst_hbm.at[pl.ds(base, SD)], idxd)
        pltpu.sync_copy(src_hbm.at[pl.ds(base, SD)], idxs)
        cps = []
        for g in range(SD // C):
            sl = pl.ds(g * C, C)
            cps += [
                pltpu.async_copy(px_hbm.at[idxd.at[sl]], gdx.at[sl], sem),
                pltpu.async_copy(py_hbm.at[idxd.at[sl]], gdy.at[sl], sem),
                pltpu.async_copy(pz_hbm.at[idxd.at[sl]], gdz.at[sl], sem),
                pltpu.async_copy(px_hbm.at[idxs.at[sl]], gsx.at[sl], sem),
                pltpu.async_copy(py_hbm.at[idxs.at[sl]], gsy.at[sl], sem),
                pltpu.async_copy(pz_hbm.at[idxs.at[sl]], gsz.at[sl], sem),
            ]
        pltpu.sync_copy(ox_hbm.at[pl.ds(base, SD)], oxv)
        pltpu.sync_copy(oy_hbm.at[pl.ds(base, SD)], oyv)
        pltpu.sync_copy(oz_hbm.at[pl.ds(base, SD)], ozv)
        for cp in cps:
            cp.wait()

        def body(g, _):
            s = pl.ds(g * 16, 16)
            dx = gdx[s] + oxv[s] - gsx[s]
            dy = gdy[s] + oyv[s] - gsy[s]
            dz = gdz[s] + ozv[s] - gsz[s]
            d2v[s] = dx * dx + dy * dy + dz * dz
            return 0

        lax.fori_loop(0, SD // 16, body, 0)
        pltpu.sync_copy(d2v, out_hbm.at[pl.ds(base, SD)])
        return 0

    lax.fori_loop(0, EW // SD, chunk_body, 0)


def _nf_body(d2_ref, wn_ref, bias_ref, nf0_ref, nf1_ref, nf2_ref):
    d = jnp.sqrt(d2_ref[...])
    cent = lax.broadcasted_iota(jnp.int32, (1, NBRP), 1).astype(jnp.float32) * 0.2
    g = jnp.exp(-((d - cent) ** 2) * 25.0)
    o = jnp.dot(g, wn_ref[...], preferred_element_type=jnp.float32)
    o = o + bias_ref[...]
    nf0_ref[...] = o[:, 0:128]
    nf1_ref[...] = o[:, 128:256]
    nf2_ref[...] = o[:, 256:384]


_EB = 512


def _nf(d2c, wn, bias):
    return pl.pallas_call(
        _nf_body,
        grid=(E // _EB,),
        in_specs=[
            pl.BlockSpec((_EB, 1), lambda i: (i, 0)),
            pl.BlockSpec((NBRP, 3 * 128), lambda i: (0, 0)),
            pl.BlockSpec((1, 3 * 128), lambda i: (0, 0)),
        ],
        out_specs=[
            pl.BlockSpec((_EB, 128), lambda i: (i, 0)),
            pl.BlockSpec((_EB, 128), lambda i: (i, 0)),
            pl.BlockSpec((_EB, 128), lambda i: (i, 0)),
        ],
        out_shape=[jax.ShapeDtypeStruct((E, 128), jnp.float32)] * 3,
    )(d2c, wn, bias)


def _aproj_body(atom_ref, wd_ref, ws_ref, ad_ref, as_ref):
    x = atom_ref[...]
    ad_ref[...] = jnp.dot(x, wd_ref[...], preferred_element_type=jnp.float32)
    as_ref[...] = jnp.dot(x, ws_ref[...], preferred_element_type=jnp.float32)


def _aproj(atom, wd, ws):
    nb = 2000
    return pl.pallas_call(
        _aproj_body,
        grid=(N // nb,),
        in_specs=[
            pl.BlockSpec((nb, H), lambda i: (i, 0)),
            pl.BlockSpec((H, 128), lambda i: (0, 0)),
            pl.BlockSpec((H, 128), lambda i: (0, 0)),
        ],
        out_specs=[
            pl.BlockSpec((nb, 128), lambda i: (i, 0)),
            pl.BlockSpec((nb, 128), lambda i: (i, 0)),
        ],
        out_shape=[jax.ShapeDtypeStruct((N, 128), jnp.float32)] * 2,
    )(atom, wd, ws)


@functools.partial(
    pl.kernel,
    out_type=[
        jax.ShapeDtypeStruct((E, 128), jnp.float32),
        jax.ShapeDtypeStruct((NW, 2, 8, 16), jnp.float32),
    ],
    mesh=plsc.VectorSubcoreMesh(**_MESH),
    scratch_types=[
        pltpu.VMEM((2, C), jnp.int32),
        pltpu.VMEM((2, C), jnp.int32),
        pltpu.VMEM((2, C, 128), jnp.float32),
        pltpu.VMEM((2, C, 128), jnp.float32),
        pltpu.VMEM((2, C, 128), jnp.float32),
        pltpu.VMEM((2, C, 128), jnp.float32),
        pltpu.VMEM((8, 16), jnp.float32),
        pltpu.VMEM((8, 16), jnp.float32),
        pltpu.SemaphoreType.DMA,
        pltpu.SemaphoreType.DMA,
        pltpu.SemaphoreType.DMA,
        pltpu.SemaphoreType.DMA,
        pltpu.SemaphoreType.DMA,
        pltpu.SemaphoreType.DMA,
    ],
)
def _pass1(dst_hbm, src_hbm, ad_hbm, as_hbm, nf_hbm, tot_hbm, st_hbm,
           idxd, idxs, gd, gs, nfv, totv, sacc, qacc,
           si0, si1, sg0, sg1, st0, st1):
    wid = lax.axis_index("s") * 2 + lax.axis_index("c")
    si = [si0, si1]
    sg = [sg0, sg1]
    st = [st0, st1]
    z = jnp.zeros((16,), jnp.float32)
    for k in range(8):
        sacc[k] = z
        qacc[k] = z
    wbase = wid * EW

    def bofs(c):
        return pl.multiple_of(wbase + c * C, 8)

    def prefetch(c, s):
        pltpu.async_copy(dst_hbm.at[pl.ds(bofs(c), C)], idxd.at[s], si[s])
        pltpu.async_copy(src_hbm.at[pl.ds(bofs(c), C)], idxs.at[s], si[s])

    def launch(c, s):
        pltpu.make_async_copy(dst_hbm.at[pl.ds(0, C)], idxd.at[s], si[s]).wait()
        pltpu.make_async_copy(src_hbm.at[pl.ds(0, C)], idxs.at[s], si[s]).wait()
        pltpu.async_copy(ad_hbm.at[idxd.at[s]], gd.at[s], sg[s])
        pltpu.async_copy(as_hbm.at[idxs.at[s]], gs.at[s], sg[s])
        pltpu.async_copy(nf_hbm.at[pl.ds(bofs(c), C)], nfv.at[s], sg[s])

    def consume_wait(s):
        pltpu.make_async_copy(nf_hbm.at[pl.ds(0, C)], gd.at[s], sg[s]).wait()
        pltpu.make_async_copy(nf_hbm.at[pl.ds(0, C)], gs.at[s], sg[s]).wait()
        pltpu.make_async_copy(nf_hbm.at[pl.ds(0, C)], nfv.at[s], sg[s]).wait()

    def compute_store(c, s, first):
        @pl.when(jnp.logical_not(first))
        def _():
            pltpu.make_async_copy(totv.at[s], tot_hbm.at[pl.ds(0, C)],
                                  st[s]).wait()

        def edge_body(j, _):
            for k in range(8):
                sl = pl.ds(k * 16, 16)
                t = gd[s, j, sl] + gs[s, j, sl] + nfv[s, j, sl]
                totv[s, j, sl] = t
                plsc.addupdate(sacc.at[k], t)
                plsc.addupdate(qacc.at[k], t * t)
            return 0

        lax.fori_loop(0, C, edge_body, 0)
        pltpu.async_copy(totv.at[s], tot_hbm.at[pl.ds(bofs(c), C)], st[s])

    prefetch(0, 0)
    launch(0, 0)
    prefetch(1, 1)

    def body(i, _):
        c0 = i * 2
        c1 = c0 + 1
        launch(c1, 1)
        consume_wait(0)
        prefetch(c0 + 2, 0)
        compute_store(c0, 0, i == 0)
        consume_wait(1)

        @pl.when(i < (NCHUNK - 3) // 2)
        def _():
            prefetch(c1 + 2, 1)

        compute_store(c1, 1, i == 0)
        launch(c0 + 2, 0)
        return 0

    lax.fori_loop(0, (NCHUNK - 1) // 2, body, 0)
    consume_wait(0)
    compute_store(NCHUNK - 1, 0, False)
    pltpu.make_async_copy(totv.at[0], tot_hbm.at[pl.ds(0, C)], st[0]).wait()
    pltpu.make_async_copy(totv.at[1], tot_hbm.at[pl.ds(0, C)], st[1]).wait()
    pltpu.sync_copy(sacc, st_hbm.at[wid, 0])
    pltpu.sync_copy(qacc, st_hbm.at[wid, 1])


def _pass2a_body(tot_ref, sums_ref, sqs_ref, g_ref, bb_ref, msg_ref):
    s = jnp.sum(sums_ref[...], axis=0, keepdims=True)
    q = jnp.sum(sqs_ref[...], axis=0, keepdims=True)
    m = s * INV_E
    var = q * INV_E - m * m
    al = g_ref[...] * lax.rsqrt(var + EPS)
    be = bb_ref[...] - m * al
    y = tot_ref[...] * al + be
    f = y[:, :H]
    c = y[:, H:]
    sig = 1.0 / (1.0 + jnp.exp(-f))
    sp = jnp.maximum(c, 0.0) + jnp.log(1.0 + jnp.exp(-jnp.abs(c)))
    msg = sig * sp
    msg_ref[0] = msg[:, :H2]
    msg_ref[1] = msg[:, H2:]


def _pass2a(tot, sums, sqs, g, bb):
    return pl.pallas_call(
        _pass2a_body,
        grid=(E // _EB,),
        in_specs=[
            pl.BlockSpec((_EB, 128), lambda i: (i, 0)),
            pl.BlockSpec((NW, 128), lambda i: (0, 0)),
            pl.BlockSpec((NW, 128), lambda i: (0, 0)),
            pl.BlockSpec((1, 128), lambda i: (0, 0)),
            pl.BlockSpec((1, 128), lambda i: (0, 0)),
        ],
        out_specs=pl.BlockSpec((2, _EB, H2), lambda i: (0, i, 0)),
        out_shape=jax.ShapeDtypeStruct((2, E, H2), jnp.float32),
    )(tot, sums, sqs, g, bb)


@functools.partial(
    pl.kernel,
    out_type=jax.ShapeDtypeStruct((2, NP, H2), jnp.float32),
    mesh=plsc.VectorSubcoreMesh(**_MESH),
    scratch_types=[
        pltpu.VMEM((C,), jnp.int32),
        pltpu.VMEM((C, H2), jnp.float32),
        pltpu.VMEM((C, 128), jnp.float32),
        pltpu.VMEM_SHARED((NP, H2), jnp.float32),
    ],
)
def _scatter(dst_hbm, msg_hbm, znp_hbm, out_hbm, idxv, msgv, mwide, aggr_sh):
    cid = lax.axis_index("c")
    sid = lax.axis_index("s")

    @pl.when(sid == 0)
    def _():
        pltpu.sync_copy(znp_hbm.at[cid], aggr_sh)

    plsc.subcore_barrier()
    wbase = sid * EC

    def chunk_body(ci, _):
        base = pl.multiple_of(wbase + ci * C, 8)
        pltpu.sync_copy(dst_hbm.at[pl.ds(base, C)], idxv)
        pltpu.sync_copy(msg_hbm.at[cid, pl.ds(base, C)], aggr_sh.at[idxv],
                        add=True)
        return 0

    lax.fori_loop(0, NCHUNK2, chunk_body, 0)
    plsc.subcore_barrier()
    pltpu.sync_copy(aggr_sh.at[pl.ds(sid * RPT, RPT)],
                    out_hbm.at[cid, pl.ds(sid * RPT, RPT)])


def _final_body(ap_ref, atom_ref, g2_ref, b2_ref, out_ref):
    a = jnp.concatenate([ap_ref[0, :N, :], ap_ref[1, :N, :]], axis=1)
    m = jnp.mean(a, axis=0, keepdims=True)
    var = jnp.mean((a - m) ** 2, axis=0, keepdims=True)
    ag = (a - m) * lax.rsqrt(var + EPS) * g2_ref[...] + b2_ref[...]
    x = atom_ref[...] + ag
    out_ref[...] = jnp.maximum(x, 0.0) + jnp.log(1.0 + jnp.exp(-jnp.abs(x)))


def _final(ap, atom, g2, b2):
    return pl.pallas_call(
        _final_body,
        out_shape=jax.ShapeDtypeStruct((N, H), jnp.float32),
    )(ap, atom, g2, b2)


def kernel(v, pos, edges, offsets_real, W, b, bn1_g, bn1_b, bn2_g, bn2_b):
    src = edges[0].astype(jnp.int32)
    dst = edges[1].astype(jnp.int32)
    offt = offsets_real.T

    post = pos.T
    d2 = _dist2(post[0], post[1], post[2], dst, src, offt[0], offt[1], offt[2])
    d2c = d2.reshape(E, 1)

    wn = jnp.concatenate([W[l, 2 * H:] for l in range(L)], axis=1)
    wn = jnp.pad(wn, ((0, NBRP - NBR), (0, 0)))
    bias_all = jnp.concatenate([b[l] for l in range(L)])[None, :]
    nfs = _nf(d2c, wn, bias_all)

    atom = v
    for l in range(L):
        ad, as_ = _aproj(atom, W[l, :H], W[l, H:2 * H])
        tot, st = _pass1(dst, src, ad, as_, nfs[l])
        str_ = st.reshape(NW, 2, 128)
        sums = str_[:, 0]
        sqs = str_[:, 1]
        msg = _pass2a(tot, sums, sqs,
                      bn1_g[l][None, :], bn1_b[l][None, :])
        m2 = jnp.concatenate([msg[0], msg[1]], axis=1)
        aggr = jax.ops.segment_sum(m2, dst, num_segments=N)
        ap = jnp.zeros((2, NP, H2), jnp.float32)
        ap = ap.at[0, :N].set(aggr[:, :H2]).at[1, :N].set(aggr[:, H2:])
        atom = _final(ap, atom, bn2_g[l][None, :], bn2_b[l][None, :])
    return atom
